# baseline (speedup 1.0000x reference)
import jax
import jax.numpy as jnp
from jax.experimental import pallas as pl
from jax.experimental.pallas import tpu as pltpu

_B, _N, _D = 64, 512, 512
_BB = 8


def _gcn_kernel(h_ref, adj_in_ref, adj_out_ref, eb_ref, oeb_ref, gw_ref,
                gbias_ref, out_ref, adj_in_b_ref, adj_out_b_ref,
                eb_b_ref, oeb_b_ref):
    @pl.when(pl.program_id(0) == 0)
    def _():
        adj_in_b_ref[...] = adj_in_ref[...].astype(jnp.bfloat16)
        adj_out_b_ref[...] = adj_out_ref[...].astype(jnp.bfloat16)
        eb_b_ref[...] = eb_ref[...].astype(jnp.bfloat16)
        oeb_b_ref[...] = oeb_ref[...].astype(jnp.bfloat16)

    ebb = eb_b_ref[...]
    oebb = oeb_b_ref[...]
    adj_in = adj_in_b_ref[...]
    adj_out = adj_out_b_ref[...]
    gw = gw_ref[...]
    gbias = gbias_ref[...]
    for i in range(_BB):
        hb = h_ref[i].astype(jnp.bfloat16)
        g = jax.lax.dot_general(hb, gw, (((1,), (1,)), ((), ())),
                                preferred_element_type=jnp.float32)
        s = jax.nn.sigmoid(g + gbias)
        s1 = s[:, 0:1].astype(jnp.bfloat16)
        s2 = s[:, 1:2].astype(jnp.bfloat16)
        s3 = s[:, 2:3].astype(jnp.bfloat16)
        a = s1 * adj_in + s2 * adj_out
        m = jnp.dot(a, hb,
                    preferred_element_type=jnp.float32).astype(jnp.bfloat16)
        acc = m + (s1 * ebb + (s2 * oebb + s3 * hb))
        out_ref[i] = jnp.maximum(acc, jnp.bfloat16(0)).astype(jnp.float32)


def kernel(inputs, adj_in, edge_bias, gate_weight, bias_gate, adj_out,
           out_edge_bias, out_gate_weight, out_bias_gate, loop_gate):
    gw = jnp.concatenate(
        [gate_weight.T, out_gate_weight.T, loop_gate.T],
        axis=0).astype(jnp.bfloat16)
    gbias = jnp.concatenate(
        [bias_gate, out_bias_gate, jnp.zeros_like(bias_gate)], axis=1)

    grid = (_B // _BB,)
    out = pl.pallas_call(
        _gcn_kernel,
        grid=grid,
        in_specs=[
            pl.BlockSpec((_BB, _N, _D), lambda b: (b, 0, 0)),
            pl.BlockSpec((_N, _N), lambda b: (0, 0)),
            pl.BlockSpec((_N, _N), lambda b: (0, 0)),
            pl.BlockSpec((_N, _D), lambda b: (0, 0)),
            pl.BlockSpec((_N, _D), lambda b: (0, 0)),
            pl.BlockSpec((3, _D), lambda b: (0, 0)),
            pl.BlockSpec((_N, 3), lambda b: (0, 0)),
        ],
        out_specs=pl.BlockSpec((_BB, _N, _D), lambda b: (b, 0, 0)),
        out_shape=jax.ShapeDtypeStruct((_B, _N, _D), jnp.float32),
        scratch_shapes=[
            pltpu.VMEM((_N, _N), jnp.bfloat16),
            pltpu.VMEM((_N, _N), jnp.bfloat16),
            pltpu.VMEM((_N, _D), jnp.bfloat16),
            pltpu.VMEM((_N, _D), jnp.bfloat16),
        ],
        compiler_params=pltpu.CompilerParams(
            dimension_semantics=("arbitrary",)),
    )(inputs, adj_in, adj_out, edge_bias, out_edge_bias, gw, gbias)
    return out


# speedup vs baseline: 1.0564x; 1.0564x over previous
import jax
import jax.numpy as jnp
from jax.experimental import pallas as pl
from jax.experimental.pallas import tpu as pltpu

_B, _N, _D = 64, 512, 512
_BB = 8


def _gcn_kernel(h_ref, adj_in_ref, adj_out_ref, eb_ref, oeb_ref, gw_ref,
                gbias_ref, out_ref, adj_cat_ref, eb_b_ref, oeb_b_ref):
    @pl.when(pl.program_id(0) == 0)
    def _():
        adj_cat_ref[0:_N, :] = adj_in_ref[...].astype(jnp.bfloat16)
        adj_cat_ref[_N:, :] = adj_out_ref[...].astype(jnp.bfloat16)
        eb_b_ref[...] = eb_ref[...].astype(jnp.bfloat16)
        oeb_b_ref[...] = oeb_ref[...].astype(jnp.bfloat16)

    ebb = eb_b_ref[...]
    oebb = oeb_b_ref[...]
    adj_cat = adj_cat_ref[...]
    gw = gw_ref[...]
    gbias = gbias_ref[...]
    for i in range(_BB):
        hb = h_ref[i].astype(jnp.bfloat16)
        g = jax.lax.dot_general(hb, gw, (((1,), (1,)), ((), ())),
                                preferred_element_type=jnp.float32)
        s = jax.nn.sigmoid(g + gbias)
        s1 = s[:, 0:1].astype(jnp.bfloat16)
        s2 = s[:, 1:2].astype(jnp.bfloat16)
        s3 = s[:, 2:3].astype(jnp.bfloat16)
        mcat = jnp.dot(adj_cat, hb,
                       preferred_element_type=jnp.float32).astype(jnp.bfloat16)
        m1 = mcat[0:_N, :]
        m2 = mcat[_N:, :]
        acc = s1 * (m1 + ebb) + (s2 * (m2 + oebb) + s3 * hb)
        out_ref[i] = jnp.maximum(acc, jnp.bfloat16(0)).astype(jnp.float32)


def kernel(inputs, adj_in, edge_bias, gate_weight, bias_gate, adj_out,
           out_edge_bias, out_gate_weight, out_bias_gate, loop_gate):
    gw = jnp.concatenate(
        [gate_weight.T, out_gate_weight.T, loop_gate.T],
        axis=0).astype(jnp.bfloat16)
    gbias = jnp.concatenate(
        [bias_gate, out_bias_gate, jnp.zeros_like(bias_gate)], axis=1)

    grid = (_B // _BB,)
    out = pl.pallas_call(
        _gcn_kernel,
        grid=grid,
        in_specs=[
            pl.BlockSpec((_BB, _N, _D), lambda b: (b, 0, 0)),
            pl.BlockSpec((_N, _N), lambda b: (0, 0)),
            pl.BlockSpec((_N, _N), lambda b: (0, 0)),
            pl.BlockSpec((_N, _D), lambda b: (0, 0)),
            pl.BlockSpec((_N, _D), lambda b: (0, 0)),
            pl.BlockSpec((3, _D), lambda b: (0, 0)),
            pl.BlockSpec((_N, 3), lambda b: (0, 0)),
        ],
        out_specs=pl.BlockSpec((_BB, _N, _D), lambda b: (b, 0, 0)),
        out_shape=jax.ShapeDtypeStruct((_B, _N, _D), jnp.float32),
        scratch_shapes=[
            pltpu.VMEM((2 * _N, _N), jnp.bfloat16),
            pltpu.VMEM((_N, _D), jnp.bfloat16),
            pltpu.VMEM((_N, _D), jnp.bfloat16),
        ],
        compiler_params=pltpu.CompilerParams(
            dimension_semantics=("arbitrary",)),
    )(inputs, adj_in, adj_out, edge_bias, out_edge_bias, gw, gbias)
    return out
